# baseline (speedup 1.0000x reference)
import functools
import jax
import jax.numpy as jnp
from jax import lax
from jax.experimental import pallas as pl
from jax.experimental.pallas import tpu as pltpu
from jax.experimental.pallas import tpu_sc as plsc

N = 10000
E = 320000
H = 128
L = 3
B = 64

NC = 2
NS = 16
NW = NC * NS
CHUNK = 128
ROWS_PER_TILE = 79
E_PAD = NW * ROWS_PER_TILE * CHUNK
ACC_ROWS = 10240
ROWS_PER_SLICE = ACC_ROWS // NS
WB = 128
WB_CHUNKS = ROWS_PER_SLICE // WB


def _seg_sum_sc_body(v_hbm, src_hbm, dst_hbm, zeros_hbm, out_hbm,
                     sidx, didx, rows, acc, gsem):
    ci = lax.axis_index("c")
    si = lax.axis_index("s")
    wid = si * NC + ci

    pltpu.sync_copy(zeros_hbm, rows)

    def zbody(k, carry):
        pltpu.sync_copy(rows.at[pl.ds(0, WB)],
                        acc.at[pl.ds(si * ROWS_PER_SLICE + k * WB, WB), :])
        return carry
    lax.fori_loop(0, WB_CHUNKS, zbody, 0)
    plsc.subcore_barrier()

    pltpu.sync_copy(src_hbm.at[wid], sidx)
    pltpu.sync_copy(dst_hbm.at[wid], didx)

    def ebody(j, carry):
        pltpu.async_copy(v_hbm.at[sidx.at[j]], rows, gsem).wait()
        pltpu.sync_copy(rows, acc.at[didx.at[j]], add=True)
        return carry
    lax.fori_loop(0, ROWS_PER_TILE, ebody, 0)
    plsc.subcore_barrier()

    def wbody(k, carry):
        r0 = si * ROWS_PER_SLICE + k * WB
        pltpu.sync_copy(acc.at[pl.ds(r0, WB), :], rows.at[pl.ds(0, WB)])
        pltpu.sync_copy(rows.at[pl.ds(0, WB)], out_hbm.at[ci, pl.ds(r0, WB), :])
        return carry
    lax.fori_loop(0, WB_CHUNKS, wbody, 0)


@functools.lru_cache(maxsize=None)
def _make_seg_sum_sc():
    return pl.kernel(
        _seg_sum_sc_body,
        mesh=plsc.VectorSubcoreMesh(core_axis_name="c", subcore_axis_name="s",
                                    num_cores=NC, num_subcores=NS),
        out_type=jax.ShapeDtypeStruct((NC, ACC_ROWS, H), jnp.float32),
        scratch_types=(
            [pltpu.VMEM((ROWS_PER_TILE, CHUNK), jnp.int32)] * 2
            + [pltpu.VMEM((CHUNK, H), jnp.float32)]
            + [pltpu.VMEM_SHARED((ACC_ROWS, H), jnp.float32)]
            + [pltpu.SemaphoreType.DMA]
        ),
    )


def _seg_sum_sc(v, src_p, dst_p, zeros):
    return _make_seg_sum_sc()(v, src_p, dst_p, zeros)


BLK = 1000
GRID = N // BLK


def _lin0_body(x_ref, w_ref, b_ref, o_ref):
    o_ref[...] = jnp.dot(x_ref[...], w_ref[...],
                         preferred_element_type=jnp.float32, precision=lax.Precision.HIGHEST) + b_ref[...]


def _lin0(x, W0, b0):
    return pl.pallas_call(
        _lin0_body,
        grid=(GRID,),
        in_specs=[
            pl.BlockSpec((BLK, H), lambda i: (i, 0)),
            pl.BlockSpec((H, H), lambda i: (0, 0)),
            pl.BlockSpec((1, H), lambda i: (0, 0)),
        ],
        out_specs=pl.BlockSpec((BLK, H), lambda i: (i, 0)),
        out_shape=jax.ShapeDtypeStruct((N, H), jnp.float32),
    )(x, W0, b0)


def _mask_body(p_ref, xm_ref, h_ref, w1a_ref, w1b_ref, b1_ref, w2t_ref, b2_ref,
               xs_ref, mb_ref):
    aggr = p_ref[0] + p_ref[1]
    hm = jnp.maximum(
        jnp.dot(aggr, w1a_ref[...], preferred_element_type=jnp.float32, precision=lax.Precision.HIGHEST)
        + jnp.dot(xm_ref[...], w1b_ref[...], preferred_element_type=jnp.float32, precision=lax.Precision.HIGHEST)
        + b1_ref[...], 0.0)
    logit = jnp.sum(hm * w2t_ref[...], axis=1, keepdims=True) + b2_ref[...]
    m = 1.0 / (1.0 + jnp.exp(-logit))
    xs_ref[...] = h_ref[...] * m
    mb_ref[...] = jnp.broadcast_to(m, mb_ref.shape)


def _mask_mlp(partials, xm, h, Wm1a, Wm1b, bm1, Wm2t, bm2):
    return pl.pallas_call(
        _mask_body,
        grid=(GRID,),
        in_specs=[
            pl.BlockSpec((NC, BLK, H), lambda i: (0, i, 0)),
            pl.BlockSpec((BLK, H), lambda i: (i, 0)),
            pl.BlockSpec((BLK, H), lambda i: (i, 0)),
            pl.BlockSpec((H, H), lambda i: (0, 0)),
            pl.BlockSpec((H, H), lambda i: (0, 0)),
            pl.BlockSpec((1, H), lambda i: (0, 0)),
            pl.BlockSpec((1, H), lambda i: (0, 0)),
            pl.BlockSpec((1, 1), lambda i: (0, 0)),
        ],
        out_specs=[
            pl.BlockSpec((BLK, H), lambda i: (i, 0)),
            pl.BlockSpec((BLK, H), lambda i: (i, 0)),
        ],
        out_shape=[
            jax.ShapeDtypeStruct((N, H), jnp.float32),
            jax.ShapeDtypeStruct((N, H), jnp.float32),
        ],
    )(partials, xm, h, Wm1a, Wm1b, bm1, Wm2t, bm2)


def _conv_body(p_ref, xs_ref, mb_ref, pt_ref, wr_ref, ws_ref, brs_ref,
               w1_ref, b1_ref, w2_ref, b2_ref,
               h_ref, xm_ref, z_ref, pooled):
    i = pl.program_id(0)
    aggr = p_ref[0] + p_ref[1]
    hn = jnp.maximum(
        jnp.dot(aggr, wr_ref[...], preferred_element_type=jnp.float32, precision=lax.Precision.HIGHEST)
        + jnp.dot(xs_ref[...], ws_ref[...], preferred_element_type=jnp.float32, precision=lax.Precision.HIGHEST)
        + brs_ref[...], 0.0)
    h_ref[...] = hn
    xm_ref[...] = hn * mb_ref[...]
    pp = jnp.dot(pt_ref[0], hn, preferred_element_type=jnp.float32, precision=lax.Precision.HIGHEST)

    @pl.when(i == 0)
    def _():
        pooled[...] = pp

    @pl.when(i > 0)
    def _():
        pooled[...] += pp

    @pl.when(i == GRID - 1)
    def _():
        z1 = jnp.maximum(
            jnp.dot(pooled[...], w1_ref[...],
                    preferred_element_type=jnp.float32, precision=lax.Precision.HIGHEST) + b1_ref[...], 0.0)
        z_ref[...] = jnp.maximum(
            jnp.dot(z1, w2_ref[...],
                    preferred_element_type=jnp.float32, precision=lax.Precision.HIGHEST) + b2_ref[...], 0.0)


def _conv_readout(partials, xs, mb, Pt, Wr, Ws, brs, W1, b1, W2, b2):
    return pl.pallas_call(
        _conv_body,
        grid=(GRID,),
        in_specs=[
            pl.BlockSpec((NC, BLK, H), lambda i: (0, i, 0)),
            pl.BlockSpec((BLK, H), lambda i: (i, 0)),
            pl.BlockSpec((BLK, H), lambda i: (i, 0)),
            pl.BlockSpec((1, B, BLK), lambda i: (i, 0, 0)),
            pl.BlockSpec((H, H), lambda i: (0, 0)),
            pl.BlockSpec((H, H), lambda i: (0, 0)),
            pl.BlockSpec((1, H), lambda i: (0, 0)),
            pl.BlockSpec((H, 2 * H), lambda i: (0, 0)),
            pl.BlockSpec((1, 2 * H), lambda i: (0, 0)),
            pl.BlockSpec((2 * H, H), lambda i: (0, 0)),
            pl.BlockSpec((1, H), lambda i: (0, 0)),
        ],
        out_specs=[
            pl.BlockSpec((BLK, H), lambda i: (i, 0)),
            pl.BlockSpec((BLK, H), lambda i: (i, 0)),
            pl.BlockSpec((B, H), lambda i: (0, 0)),
        ],
        out_shape=[
            jax.ShapeDtypeStruct((N, H), jnp.float32),
            jax.ShapeDtypeStruct((N, H), jnp.float32),
            jax.ShapeDtypeStruct((B, H), jnp.float32),
        ],
        scratch_shapes=[pltpu.VMEM((B, H), jnp.float32)],
    )(partials, xs, mb, Pt, Wr, Ws, brs, W1, b1, W2, b2)


def kernel(x, edge_index, batch, W0, b0, Wr, br, Ws, bs, Wm1, bm1, Wm2, bm2,
           W1, b1, W2, b2):
    src = edge_index[0]
    dst = edge_index[1]
    pad = E_PAD - E
    src_p = jnp.concatenate([src, jnp.zeros((pad,), jnp.int32)]
                            ).reshape(NW, ROWS_PER_TILE, CHUNK)
    pad_dst = N + jnp.arange(pad, dtype=jnp.int32) % (ACC_ROWS - N)
    dst_p = jnp.concatenate([dst, pad_dst]).reshape(NW, ROWS_PER_TILE, CHUNK)
    zeros = jnp.zeros((CHUNK, H), jnp.float32)
    Pt = (batch[None, :] == jnp.arange(B, dtype=jnp.int32)[:, None]
          ).astype(jnp.float32).reshape(B, GRID, BLK).transpose(1, 0, 2)
    b0r = b0.reshape(1, H)
    b1r = b1.reshape(1, 2 * H)
    b2r = b2.reshape(1, H)

    h = _lin0(x, W0, b0r)
    xm = h
    outs = []
    for i in range(L):
        a_m = _seg_sum_sc(xm, src_p, dst_p, zeros)
        xs, mb = _mask_mlp(a_m, xm, h,
                           Wm1[i, :H], Wm1[i, H:], bm1[i].reshape(1, H),
                           Wm2[i].reshape(1, H), bm2[i].reshape(1, 1))
        a_c = _seg_sum_sc(xs, src_p, dst_p, zeros)
        h, xm, z = _conv_readout(a_c, xs, mb, Pt, Wr[i], Ws[i],
                                 (br[i] + bs[i]).reshape(1, H),
                                 W1, b1r, W2, b2r)
        outs.append(z)
    return jnp.stack(outs, axis=0)


# speedup vs baseline: 1.0594x; 1.0594x over previous
import functools
import jax
import jax.numpy as jnp
from jax import lax
from jax.experimental import pallas as pl
from jax.experimental.pallas import tpu as pltpu
from jax.experimental.pallas import tpu_sc as plsc

N = 10000
E = 320000
H = 128
L = 3
B = 64

NC = 2
NS = 16
NW = NC * NS
CHUNK = 128
ROWS_PER_TILE = 79
E_PAD = NW * ROWS_PER_TILE * CHUNK
ACC_ROWS = 10240
ROWS_PER_SLICE = ACC_ROWS // NS
WB = 128
WB_CHUNKS = ROWS_PER_SLICE // WB


def _seg_sum_sc_body(v_hbm, src_hbm, dst_hbm, zeros_hbm, out_hbm,
                     sidx, didx, rows, acc, gsem):
    ci = lax.axis_index("c")
    si = lax.axis_index("s")
    wid = si * NC + ci

    pltpu.sync_copy(zeros_hbm, rows)

    def zbody(k, carry):
        pltpu.sync_copy(rows.at[pl.ds(0, WB)],
                        acc.at[pl.ds(si * ROWS_PER_SLICE + k * WB, WB), :])
        return carry
    lax.fori_loop(0, WB_CHUNKS, zbody, 0)
    plsc.subcore_barrier()

    pltpu.sync_copy(src_hbm.at[wid], sidx)
    pltpu.sync_copy(dst_hbm.at[wid], didx)

    def ebody(j, carry):
        pltpu.async_copy(v_hbm.at[sidx.at[j]], rows, gsem).wait()
        pltpu.sync_copy(rows, acc.at[didx.at[j]], add=True)
        return carry
    lax.fori_loop(0, ROWS_PER_TILE, ebody, 0)
    plsc.subcore_barrier()

    def wbody(k, carry):
        r0 = si * ROWS_PER_SLICE + k * WB
        pltpu.sync_copy(acc.at[pl.ds(r0, WB), :], rows.at[pl.ds(0, WB)])
        pltpu.sync_copy(rows.at[pl.ds(0, WB)], out_hbm.at[ci, pl.ds(r0, WB), :])
        return carry
    lax.fori_loop(0, WB_CHUNKS, wbody, 0)


@functools.lru_cache(maxsize=None)
def _make_seg_sum_sc():
    return pl.kernel(
        _seg_sum_sc_body,
        mesh=plsc.VectorSubcoreMesh(core_axis_name="c", subcore_axis_name="s",
                                    num_cores=NC, num_subcores=NS),
        out_type=jax.ShapeDtypeStruct((NC, ACC_ROWS, H), jnp.float32),
        scratch_types=(
            [pltpu.VMEM((ROWS_PER_TILE, CHUNK), jnp.int32)] * 2
            + [pltpu.VMEM((CHUNK, H), jnp.float32)]
            + [pltpu.VMEM_SHARED((ACC_ROWS, H), jnp.float32)]
            + [pltpu.SemaphoreType.DMA]
        ),
    )


def _seg_sum_sc(v, src_p, dst_p, zeros):
    return _make_seg_sum_sc()(v, src_p, dst_p, zeros)


BLK = 1000
GRID = N // BLK


def _lin0_body(x_ref, w_ref, b_ref, o_ref):
    o_ref[...] = jnp.dot(x_ref[...], w_ref[...],
                         preferred_element_type=jnp.float32) + b_ref[...]


def _lin0(x, W0, b0):
    return pl.pallas_call(
        _lin0_body,
        grid=(GRID,),
        in_specs=[
            pl.BlockSpec((BLK, H), lambda i: (i, 0)),
            pl.BlockSpec((H, H), lambda i: (0, 0)),
            pl.BlockSpec((1, H), lambda i: (0, 0)),
        ],
        out_specs=pl.BlockSpec((BLK, H), lambda i: (i, 0)),
        out_shape=jax.ShapeDtypeStruct((N, H), jnp.float32),
    )(x, W0, b0)


def _mask_body(p_ref, xm_ref, h_ref, w1a_ref, w1b_ref, b1_ref, w2t_ref, b2_ref,
               xs_ref, mb_ref):
    aggr = p_ref[0] + p_ref[1]
    hm = jnp.maximum(
        jnp.dot(aggr, w1a_ref[...], preferred_element_type=jnp.float32)
        + jnp.dot(xm_ref[...], w1b_ref[...], preferred_element_type=jnp.float32)
        + b1_ref[...], 0.0)
    logit = jnp.dot(hm, w2t_ref[...],
                    preferred_element_type=jnp.float32) + b2_ref[...]
    m = 1.0 / (1.0 + jnp.exp(-logit))
    xs_ref[...] = h_ref[...] * m
    mb_ref[...] = jnp.broadcast_to(m, mb_ref.shape)


def _mask_mlp(partials, xm, h, Wm1a, Wm1b, bm1, Wm2t, bm2):
    return pl.pallas_call(
        _mask_body,
        grid=(GRID,),
        in_specs=[
            pl.BlockSpec((NC, BLK, H), lambda i: (0, i, 0)),
            pl.BlockSpec((BLK, H), lambda i: (i, 0)),
            pl.BlockSpec((BLK, H), lambda i: (i, 0)),
            pl.BlockSpec((H, H), lambda i: (0, 0)),
            pl.BlockSpec((H, H), lambda i: (0, 0)),
            pl.BlockSpec((1, H), lambda i: (0, 0)),
            pl.BlockSpec((H, 1), lambda i: (0, 0)),
            pl.BlockSpec((1, 1), lambda i: (0, 0)),
        ],
        out_specs=[
            pl.BlockSpec((BLK, H), lambda i: (i, 0)),
            pl.BlockSpec((BLK, H), lambda i: (i, 0)),
        ],
        out_shape=[
            jax.ShapeDtypeStruct((N, H), jnp.float32),
            jax.ShapeDtypeStruct((N, H), jnp.float32),
        ],
    )(partials, xm, h, Wm1a, Wm1b, bm1, Wm2t, bm2)


def _conv_body(p_ref, xs_ref, mb_ref, pt_ref, wr_ref, ws_ref, brs_ref,
               w1_ref, b1_ref, w2_ref, b2_ref,
               h_ref, xm_ref, z_ref, pooled):
    i = pl.program_id(0)
    aggr = p_ref[0] + p_ref[1]
    hn = jnp.maximum(
        jnp.dot(aggr, wr_ref[...], preferred_element_type=jnp.float32)
        + jnp.dot(xs_ref[...], ws_ref[...], preferred_element_type=jnp.float32)
        + brs_ref[...], 0.0)
    h_ref[...] = hn
    xm_ref[...] = hn * mb_ref[...]
    pp = jnp.dot(pt_ref[0], hn, preferred_element_type=jnp.float32)

    @pl.when(i == 0)
    def _():
        pooled[...] = pp

    @pl.when(i > 0)
    def _():
        pooled[...] += pp

    @pl.when(i == GRID - 1)
    def _():
        z1 = jnp.maximum(
            jnp.dot(pooled[...], w1_ref[...],
                    preferred_element_type=jnp.float32) + b1_ref[...], 0.0)
        z_ref[...] = jnp.maximum(
            jnp.dot(z1, w2_ref[...],
                    preferred_element_type=jnp.float32) + b2_ref[...], 0.0)


def _conv_readout(partials, xs, mb, Pt, Wr, Ws, brs, W1, b1, W2, b2):
    return pl.pallas_call(
        _conv_body,
        grid=(GRID,),
        in_specs=[
            pl.BlockSpec((NC, BLK, H), lambda i: (0, i, 0)),
            pl.BlockSpec((BLK, H), lambda i: (i, 0)),
            pl.BlockSpec((BLK, H), lambda i: (i, 0)),
            pl.BlockSpec((1, B, BLK), lambda i: (i, 0, 0)),
            pl.BlockSpec((H, H), lambda i: (0, 0)),
            pl.BlockSpec((H, H), lambda i: (0, 0)),
            pl.BlockSpec((1, H), lambda i: (0, 0)),
            pl.BlockSpec((H, 2 * H), lambda i: (0, 0)),
            pl.BlockSpec((1, 2 * H), lambda i: (0, 0)),
            pl.BlockSpec((2 * H, H), lambda i: (0, 0)),
            pl.BlockSpec((1, H), lambda i: (0, 0)),
        ],
        out_specs=[
            pl.BlockSpec((BLK, H), lambda i: (i, 0)),
            pl.BlockSpec((BLK, H), lambda i: (i, 0)),
            pl.BlockSpec((B, H), lambda i: (0, 0)),
        ],
        out_shape=[
            jax.ShapeDtypeStruct((N, H), jnp.float32),
            jax.ShapeDtypeStruct((N, H), jnp.float32),
            jax.ShapeDtypeStruct((B, H), jnp.float32),
        ],
        scratch_shapes=[pltpu.VMEM((B, H), jnp.float32)],
    )(partials, xs, mb, Pt, Wr, Ws, brs, W1, b1, W2, b2)


def kernel(x, edge_index, batch, W0, b0, Wr, br, Ws, bs, Wm1, bm1, Wm2, bm2,
           W1, b1, W2, b2):
    src = edge_index[0]
    dst = edge_index[1]
    pad = E_PAD - E
    src_p = jnp.concatenate([src, jnp.zeros((pad,), jnp.int32)]
                            ).reshape(NW, ROWS_PER_TILE, CHUNK)
    pad_dst = N + jnp.arange(pad, dtype=jnp.int32) % (ACC_ROWS - N)
    dst_p = jnp.concatenate([dst, pad_dst]).reshape(NW, ROWS_PER_TILE, CHUNK)
    zeros = jnp.zeros((CHUNK, H), jnp.float32)
    Pt = (batch[None, :] == jnp.arange(B, dtype=jnp.int32)[:, None]
          ).astype(jnp.float32).reshape(B, GRID, BLK).transpose(1, 0, 2)
    b0r = b0.reshape(1, H)
    b1r = b1.reshape(1, 2 * H)
    b2r = b2.reshape(1, H)

    h = _lin0(x, W0, b0r)
    xm = h
    outs = []
    for i in range(L):
        a_m = _seg_sum_sc(xm, src_p, dst_p, zeros)
        xs, mb = _mask_mlp(a_m, xm, h,
                           Wm1[i, :H], Wm1[i, H:], bm1[i].reshape(1, H),
                           Wm2[i], bm2[i].reshape(1, 1))
        a_c = _seg_sum_sc(xs, src_p, dst_p, zeros)
        h, xm, z = _conv_readout(a_c, xs, mb, Pt, Wr[i], Ws[i],
                                 (br[i] + bs[i]).reshape(1, H),
                                 W1, b1r, W2, b2r)
        outs.append(z)
    return jnp.stack(outs, axis=0)
